# Initial kernel scaffold; baseline (speedup 1.0000x reference)
import jax
import jax.numpy as jnp
from jax.experimental import pallas as pl


def kernel(x, w1_t, b1_2d, w2_t, b2_2d):
    raise NotImplementedError("write your pallas kernel here")


# speedup vs baseline: 1.0298x; 1.0298x over previous
import functools

import jax
import jax.numpy as jnp
from jax.experimental import pallas as pl
from jax.experimental.pallas import tpu as pltpu


def _fused_kernel(x_ref, w1_ref, b1_ref, w2_ref, b2_ref, out_ref, *, inv_hw):
    s = jnp.sum(x_ref[0], axis=-1, keepdims=True)

    h = jax.lax.dot_general(
        s, w1_ref[...],
        dimension_numbers=(((0,), (0,)), ((), ())),
        preferred_element_type=jnp.float32,
    )
    h = jnp.maximum(h * inv_hw + b1_ref[...], 0.0)

    o = jnp.dot(h, w2_ref[...],
                preferred_element_type=jnp.float32) + b2_ref[...]
    out_ref[0] = 2.0 * jax.nn.sigmoid(o) - 1.0


def kernel(x, w1_t, b1_2d, w2_t, b2_2d):
    n, c = x.shape[0], x.shape[1]
    hw = 1
    for d in x.shape[2:]:
        hw *= d
    x_flat = x.reshape(n, c, hw)
    hidden = w1_t.shape[1]
    out_dim = w2_t.shape[1]

    body = functools.partial(_fused_kernel, inv_hw=1.0 / float(hw))

    itemsize = x_flat.dtype.itemsize
    cost = pl.CostEstimate(
        flops=int(n) * int(c) * (int(hw) + 2 * int(hidden))
              + 2 * int(n) * int(hidden) * int(out_dim),
        transcendentals=int(n) * int(out_dim),
        bytes_accessed=int(n) * int(c) * int(hw) * int(itemsize)
                       + int(n) * int(out_dim) * 4,
    )

    out3 = pl.pallas_call(
        body,
        out_shape=jax.ShapeDtypeStruct((n, 1, out_dim), jnp.float32),
        grid=(n,),
        in_specs=[
            pl.BlockSpec((1, c, hw), lambda i: (i, 0, 0)),
            pl.BlockSpec((c, hidden), lambda i: (0, 0)),
            pl.BlockSpec((1, hidden), lambda i: (0, 0)),
            pl.BlockSpec((hidden, out_dim), lambda i: (0, 0)),
            pl.BlockSpec((1, out_dim), lambda i: (0, 0)),
        ],
        out_specs=pl.BlockSpec((1, 1, out_dim), lambda i: (i, 0, 0)),
        compiler_params=pltpu.CompilerParams(
            dimension_semantics=("parallel",),
            vmem_limit_bytes=48 * 1024 * 1024,
        ),
        cost_estimate=cost,
    )(x_flat, w1_t, b1_2d, w2_t, b2_2d)

    return out3.reshape(n, out_dim)
